# baseline (speedup 1.0000x reference)
import functools

import jax
import jax.numpy as jnp
from jax import lax
from jax.experimental import pallas as pl
from jax.experimental.pallas import tpu as pltpu
from jax.experimental.pallas import tpu_sc as plsc

NB = 1000000
D = 32
B = 16384
NC = 2
NS = 16
NW = NC * NS
BPW = B // NW
CHUNK = 128
NCH = BPW // CHUNK
L = 16


def _body(ii_hbm, jj_hbm, te_hbm, ce_hbm, tb_hbm, cb_hbm, out_hbm,
          ii_v, jj_v, a_v, b_v, tb_v, cb_v, out_v, sem):
    wid = lax.axis_index("s") * NC + lax.axis_index("c")
    base = wid * BPW

    for c in range(NCH):
        pltpu.sync_copy(ii_hbm.at[wid * NCH + c], ii_v.at[c])
        pltpu.sync_copy(jj_hbm.at[wid * NCH + c], jj_v.at[c])

    handles = []
    for c in range(NCH):
        sl = pl.ds(c * CHUNK, CHUNK)
        handles.append(pltpu.async_copy(te_hbm.at[ii_v.at[c]], a_v.at[sl], sem))
        handles.append(pltpu.async_copy(ce_hbm.at[jj_v.at[c]], b_v.at[sl], sem))
        handles.append(pltpu.async_copy(tb_hbm.at[ii_v.at[c]], tb_v.at[sl], sem))
        handles.append(pltpu.async_copy(cb_hbm.at[jj_v.at[c]], cb_v.at[sl], sem))
    for h in handles:
        h.wait()

    iota = lax.iota(jnp.int32, L)

    def g_body(g, carry):
        rows = g * L + iota
        acc = tb_v[pl.ds(g * L, L)] + cb_v[pl.ds(g * L, L)]
        for d in range(D):
            dcol = jnp.full((L,), d, jnp.int32)
            va = plsc.load_gather(a_v, [rows, dcol])
            vb = plsc.load_gather(b_v, [rows, dcol])
            acc = acc + va * vb
        out_v[pl.ds(g * L, L)] = acc
        return carry

    lax.fori_loop(0, BPW // L, g_body, 0)

    pltpu.sync_copy(out_v, out_hbm.at[pl.ds(base, BPW)])


@functools.partial(jax.jit, static_argnames=())
def _run(ii, jj, te, ce, tb, cb):
    mesh = plsc.VectorSubcoreMesh(core_axis_name="c", subcore_axis_name="s")
    k = functools.partial(
        pl.kernel,
        mesh=mesh,
        compiler_params=pltpu.CompilerParams(
            needs_layout_passes=False, use_tc_tiling_on_sc=False),
        out_type=jax.ShapeDtypeStruct((B,), jnp.float32),
        scratch_types=[
            pltpu.VMEM((NCH, CHUNK), jnp.int32),
            pltpu.VMEM((NCH, CHUNK), jnp.int32),
            pltpu.VMEM((BPW, D), jnp.float32),
            pltpu.VMEM((BPW, D), jnp.float32),
            pltpu.VMEM((BPW,), jnp.float32),
            pltpu.VMEM((BPW,), jnp.float32),
            pltpu.VMEM((BPW,), jnp.float32),
            pltpu.SemaphoreType.DMA,
        ],
    )(_body)
    return k(ii, jj, te, ce, tb, cb)


def kernel(pair, target_emb, context_emb, target_bias, context_bias):
    ii = pair[:, 0].astype(jnp.int32).reshape(B // CHUNK, CHUNK)
    jj = pair[:, 1].astype(jnp.int32).reshape(B // CHUNK, CHUNK)
    tb = target_bias.reshape(-1)
    cb = context_bias.reshape(-1)
    out = _run(ii, jj, target_emb, context_emb, tb, cb)
    return out.reshape(B, 1)
